# Initial kernel scaffold; baseline (speedup 1.0000x reference)
import jax
import jax.numpy as jnp
from jax.experimental import pallas as pl


def kernel(xyz, W1, b1, W2, b2):
    raise NotImplementedError("write your pallas kernel here")


# speedup vs baseline: 1.9174x; 1.9174x over previous
import functools

import jax
import jax.numpy as jnp
from jax import lax
from jax.experimental import pallas as pl

RADIUS2 = 0.1 * 0.1
NS = 32


def _lse_body(xi_ref, xyz_ref, xyzT_ref, w1_ref, b1_ref, w2_ref, b2_ref,
              out_ref, *, M, K):
    N = xyz_ref.shape[1]
    xi = xi_ref[0]
    w1 = w1_ref[...]
    b1 = b1_ref[...]
    w2 = w2_ref[...]
    b2 = b2_ref[...]

    u_i = lax.dot_general(xi, w1, (((1,), (1,)), ((), ())),
                          preferred_element_type=jnp.float32)

    xi0 = xi[:, 0].reshape(M, 1)
    xi1 = xi[:, 1].reshape(M, 1)
    xi2 = xi[:, 2].reshape(M, 1)

    rk = lax.broadcasted_iota(jnp.int32, (K, K), 0)
    ck = lax.broadcasted_iota(jnp.int32, (K, K), 1)
    lt = (rk < ck).astype(jnp.float32)

    nchunks = N // K

    def chunk_step(c, carry):
        feat, cnt, rank_off = carry
        xj0 = xyzT_ref[0, 0, pl.ds(c * K, K)].reshape(1, K)
        xj1 = xyzT_ref[0, 1, pl.ds(c * K, K)].reshape(1, K)
        xj2 = xyzT_ref[0, 2, pl.ds(c * K, K)].reshape(1, K)
        d0 = xi0 - xj0
        d1 = xi1 - xj1
        d2c = xi2 - xj2
        d2 = (d0 * d0 + d1 * d1) + d2c * d2c
        win = d2 < RADIUS2
        winf = win.astype(jnp.float32)
        chunk_cnt = jnp.sum(winf, axis=1, keepdims=True)
        rank_ex = rank_off + lax.dot_general(
            winf, lt, (((1,), (0,)), ((), ())),
            preferred_element_type=jnp.float32)
        sel = jnp.logical_and(win, rank_ex < float(NS) - 0.5)

        xj_rows = xyz_ref[0, pl.ds(c * K, K), :]
        u_j = lax.dot_general(xj_rows, w1, (((1,), (1,)), ((), ())),
                              preferred_element_type=jnp.float32)
        h1 = jax.nn.relu(u_j[None, :, :] - u_i[:, None, :] + b1)
        h2 = lax.dot_general(h1.reshape(M * K, 32), w2,
                             (((1,), (1,)), ((), ())),
                             preferred_element_type=jnp.float32)
        h2 = jax.nn.relu(h2 + b2).reshape(M, K, 32)
        h2 = h2 * sel.astype(jnp.float32)[:, :, None]
        feat = jnp.maximum(feat, jnp.max(h2, axis=1))
        return feat, cnt + chunk_cnt, rank_off + chunk_cnt

    feat0 = jnp.zeros((M, 32), jnp.float32)
    cnt0 = jnp.zeros((M, 1), jnp.float32)
    feat, cnt, _ = lax.fori_loop(0, nchunks, chunk_step, (feat0, cnt0, cnt0))
    density = jnp.minimum(cnt, float(NS)) * (1.0 / float(NS))
    out_ref[0] = jnp.concatenate([feat, density], axis=1)


def kernel(xyz, W1, b1, W2, b2):
    B, N, _ = xyz.shape
    M = 128
    K = 128
    xyzT = jnp.transpose(xyz, (0, 2, 1))

    body = functools.partial(_lse_body, M=M, K=K)
    out = pl.pallas_call(
        body,
        grid=(B, N // M),
        in_specs=[
            pl.BlockSpec((1, M, 3), lambda b, m: (b, m, 0)),
            pl.BlockSpec((1, N, 3), lambda b, m: (b, 0, 0)),
            pl.BlockSpec((1, 3, N), lambda b, m: (b, 0, 0)),
            pl.BlockSpec((32, 3), lambda b, m: (0, 0)),
            pl.BlockSpec((32,), lambda b, m: (0,)),
            pl.BlockSpec((32, 32), lambda b, m: (0, 0)),
            pl.BlockSpec((32,), lambda b, m: (0,)),
        ],
        out_specs=pl.BlockSpec((1, M, 33), lambda b, m: (b, m, 0)),
        out_shape=jax.ShapeDtypeStruct((B, N, 33), jnp.float32),
    )(xyz, xyz, xyzT, W1, b1, W2, b2)
    return out


# speedup vs baseline: 29.5017x; 15.3860x over previous
import functools

import jax
import jax.numpy as jnp
from jax import lax
from jax.experimental import pallas as pl
from jax.experimental.pallas import tpu as pltpu
from jax.experimental.pallas import tpu_sc as plsc

RADIUS2 = 0.1 * 0.1
NS = 32
NWORKERS = 32
LANES = 16


def _mask_pack_body(xi_ref, xyzT_ref, out_ref, *, MA):
    N = xyzT_ref.shape[2]
    xi = xi_ref[0]
    xi0 = xi[:, 0].reshape(MA, 1)
    xi1 = xi[:, 1].reshape(MA, 1)
    xi2 = xi[:, 2].reshape(MA, 1)

    rj = lax.broadcasted_iota(jnp.int32, (128, 8), 0)
    cw = lax.broadcasted_iota(jnp.int32, (128, 8), 1)
    pmat = jnp.where((rj >> 4) == cw, 1 << (rj & 15), 0).astype(jnp.float32)

    for c in range(N // 128):
        xj0 = xyzT_ref[0, 0, c * 128:(c + 1) * 128].reshape(1, 128)
        xj1 = xyzT_ref[0, 1, c * 128:(c + 1) * 128].reshape(1, 128)
        xj2 = xyzT_ref[0, 2, c * 128:(c + 1) * 128].reshape(1, 128)
        d0 = xi0 - xj0
        d1 = xi1 - xj1
        d2c = xi2 - xj2
        d2 = (d0 * d0 + d1 * d1) + d2c * d2c
        winf = (d2 < RADIUS2).astype(jnp.float32)
        words8 = lax.dot_general(winf, pmat, (((1,), (0,)), ((), ())),
                                 preferred_element_type=jnp.float32)
        out_ref[0, :, c, :] = words8.astype(jnp.int32)


def _sc_select_body(words_hbm, xyzT_hbm, relout_hbm, dens_hbm,
                    wbuf, xv, yv, zv, nzv, nzb, selidx, staging, densbuf,
                    *, N, CP):
    total = dens_hbm.shape[0]
    per_w = total // NWORKERS
    nchunk = per_w // CP
    cid = lax.axis_index("c")
    sid = lax.axis_index("s")
    wid = sid * 2 + cid
    base = wid * per_w
    b = base // N

    pltpu.sync_copy(xyzT_hbm.at[pl.ds((b * 3 + 0) * N, N)], xv)
    pltpu.sync_copy(xyzT_hbm.at[pl.ds((b * 3 + 1) * N, N)], yv)
    pltpu.sync_copy(xyzT_hbm.at[pl.ds((b * 3 + 2) * N, N)], zv)

    iota = lax.iota(jnp.int32, LANES)
    zf = jnp.zeros((LANES,), jnp.float32)

    def zero_body(i, carry):
        staging[pl.ds(i * LANES, LANES)] = zf
        return carry
    lax.fori_loop(0, (CP * 128) // LANES, zero_body, 0)

    def chunk_body(ch, carry):
        g0 = base + ch * CP
        pltpu.sync_copy(words_hbm.at[pl.ds(g0 * 256, CP * 256)], wbuf)

        def point_body(pi, carry2):
            p_loc = g0 + pi - b * N
            fill = jnp.full((LANES,), p_loc, jnp.int32)
            selidx[pl.ds(0, LANES)] = fill
            selidx[pl.ds(16, LANES)] = fill
            selidx[pl.ds(32, LANES)] = fill

            off = jnp.int32(0)
            for w in range(16):
                v = wbuf[pl.ds(pi * 256 + w * 16, LANES)]
                m = v != 0
                mi = m.astype(jnp.int32)
                pos = jnp.full((LANES,), off - 1, jnp.int32) + plsc.cumsum(mi)
                plsc.store_scatter(nzv, [pos], v, mask=m)
                plsc.store_scatter(nzb, [pos],
                                   jnp.full((LANES,), w * 16, jnp.int32) + iota,
                                   mask=m)
                off = off + jnp.sum(mi)

            def word_body(k, carry3):
                nsel, cnt = carry3
                kf = jnp.full((LANES,), k, jnp.int32)
                wv = plsc.load_gather(nzv, [kf])
                wb = plsc.load_gather(nzb, [kf])
                bits = ((wv >> iota) & 1) == 1
                bi = bits.astype(jnp.int32)
                pc2 = jnp.sum(bi)
                slots = jnp.full((LANES,), nsel - 1, jnp.int32) + plsc.cumsum(bi)

                @pl.when(nsel < NS)
                def _():
                    plsc.store_scatter(selidx, [slots], wb * 16 + iota,
                                       mask=bits)
                return nsel + pc2, cnt + pc2

            _, cnt = lax.fori_loop(0, off, word_body,
                                   (jnp.int32(0), jnp.int32(0)))

            pf = jnp.full((LANES,), p_loc, jnp.int32)
            px = plsc.load_gather(xv, [pf])
            py = plsc.load_gather(yv, [pf])
            pz = plsc.load_gather(zv, [pf])
            for s in range(2):
                idxv = selidx[pl.ds(s * LANES, LANES)]
                for coord, (buf, pc) in enumerate(((xv, px), (yv, py), (zv, pz))):
                    g = plsc.load_gather(buf, [idxv])
                    rel = g - pc
                    pos = iota * 4 + jnp.full((LANES,), pi * 128 + 64 * s + coord,
                                              jnp.int32)
                    plsc.store_scatter(staging, [pos], rel)

            dens = jnp.minimum(cnt, NS).astype(jnp.float32) * (1.0 / NS)
            plsc.store_scatter(densbuf, [jnp.full((LANES,), pi, jnp.int32)],
                               jnp.full((LANES,), dens, jnp.float32),
                               mask=iota == 0)
            return carry2

        lax.fori_loop(0, CP, point_body, 0)
        pltpu.sync_copy(staging, relout_hbm.at[pl.ds(g0 * 128, CP * 128)])
        pltpu.sync_copy(densbuf, dens_hbm.at[pl.ds(g0, CP)])
        return carry

    lax.fori_loop(0, nchunk, chunk_body, 0)


def _mlp_pool_body(rel_ref, w1e_ref, b1e_ref, w2e_ref, b2e_ref, out_ref):
    rel = rel_ref[...]
    h1 = jax.nn.relu(
        lax.dot_general(rel, w1e_ref[...], (((1,), (0,)), ((), ())),
                        preferred_element_type=jnp.float32) + b1e_ref[...])
    h2 = jax.nn.relu(
        lax.dot_general(h1, w2e_ref[...], (((1,), (0,)), ((), ())),
                        preferred_element_type=jnp.float32) + b2e_ref[...])
    m = h2
    for half in (512, 256, 128, 64, 32):
        m = jnp.maximum(m[:, :half], m[:, half:])
    out_ref[...] = m


def kernel(xyz, W1, b1, W2, b2):
    B, N, _ = xyz.shape
    BN = B * N
    xyzT = jnp.transpose(xyz, (0, 2, 1))

    MA = 256
    words = pl.pallas_call(
        functools.partial(_mask_pack_body, MA=MA),
        grid=(B, N // MA),
        in_specs=[
            pl.BlockSpec((1, MA, 3), lambda b_, m_: (b_, m_, 0)),
            pl.BlockSpec((1, 3, N), lambda b_, m_: (b_, 0, 0)),
        ],
        out_specs=pl.BlockSpec((1, MA, 32, 8), lambda b_, m_: (b_, m_, 0, 0)),
        out_shape=jax.ShapeDtypeStruct((B, N, 32, 8), jnp.int32),
    )(xyz, xyzT)
    words_flat = words.reshape(BN * 256)

    CP = 32
    mesh = plsc.VectorSubcoreMesh(core_axis_name="c", subcore_axis_name="s")
    sc_select = functools.partial(
        pl.kernel,
        functools.partial(_sc_select_body, N=N, CP=CP),
        out_type=[jax.ShapeDtypeStruct((BN * 128,), jnp.float32),
                  jax.ShapeDtypeStruct((BN,), jnp.float32)],
        mesh=mesh,
        compiler_params=pltpu.CompilerParams(needs_layout_passes=False),
        scratch_types=[
            pltpu.VMEM((CP * 256,), jnp.int32),
            pltpu.VMEM((N,), jnp.float32),
            pltpu.VMEM((N,), jnp.float32),
            pltpu.VMEM((N,), jnp.float32),
            pltpu.VMEM((272,), jnp.int32),
            pltpu.VMEM((272,), jnp.int32),
            pltpu.VMEM((48,), jnp.int32),
            pltpu.VMEM((CP * 128,), jnp.float32),
            pltpu.VMEM((CP,), jnp.float32),
        ],
    )()
    rel_flat, dens = sc_select(words_flat, xyzT.reshape(B * 3 * N))
    rel = rel_flat.reshape(BN, 128)

    w1pad = jnp.concatenate([W1, jnp.zeros((32, 1), jnp.float32)], axis=1)
    eye32 = jnp.eye(32, dtype=jnp.float32)
    w1e = jnp.kron(eye32, w1pad.T)
    w2e = jnp.kron(eye32, W2.T)
    b1e = jnp.tile(b1, NS)
    b2e = jnp.tile(b2, NS)

    PC = 512
    feat = pl.pallas_call(
        _mlp_pool_body,
        grid=(BN // PC,),
        in_specs=[
            pl.BlockSpec((PC, 128), lambda i: (i, 0)),
            pl.BlockSpec((128, 1024), lambda i: (0, 0)),
            pl.BlockSpec((1024,), lambda i: (0,)),
            pl.BlockSpec((1024, 1024), lambda i: (0, 0)),
            pl.BlockSpec((1024,), lambda i: (0,)),
        ],
        out_specs=pl.BlockSpec((PC, 32), lambda i: (i, 0)),
        out_shape=jax.ShapeDtypeStruct((BN, 32), jnp.float32),
    )(rel, w1e, b1e, w2e, b2e)

    return jnp.concatenate(
        [feat.reshape(B, N, 32), dens.reshape(B, N, 1)], axis=-1)
